# baseline (speedup 1.0000x reference)
import functools

import jax
import jax.numpy as jnp
from jax.experimental import pallas as pl
from jax.experimental.pallas import tpu as pltpu

_BLK = 1024
_CBLK = 16384
_KEY_LO = 0x3DFFFF00
_KEY_HI = 0x3F800000


def _router_body(cap, nblk, ncblk, x_ref, w_ref,
                 gates_ref, disp_ref, loss_ref,
                 keysT, pmax, zacc, pacc, thri, thrf, runeq, runf):
    i = pl.program_id(0)
    e_dim = w_ref.shape[0]
    blk = x_ref.shape[0]
    cblk = gates_ref.shape[0]

    @pl.when(i < nblk)
    def _phase_logits():
        xb = x_ref[...]
        w = w_ref[...]
        lg = jax.lax.dot_general(
            w, xb, (((1,), (1,)), ((), ())),
            preferred_element_type=jnp.float32)
        m = jnp.max(lg, axis=0, keepdims=True)
        ex = jnp.exp(lg - m)
        s = jnp.sum(ex, axis=0, keepdims=True)
        pr = ex / s
        lse = m + jnp.log(s)
        z_part = lse * lse

        @pl.when(i == 0)
        def _():
            zacc[...] = z_part
            pacc[...] = pr

        @pl.when(i > 0)
        def _():
            zacc[...] += z_part
            pacc[...] += pr

        pm = jnp.max(pr, axis=0, keepdims=True)
        eidx = jax.lax.broadcasted_iota(jnp.int32, (e_dim, blk), 0)
        cand = jnp.where(pr == pm, eidx, e_dim)
        amin = jnp.min(cand, axis=0, keepdims=True)
        kb = jax.lax.bitcast_convert_type(pr, jnp.int32)
        keys = jnp.where(eidx == amin, kb, 0)
        c0 = pl.multiple_of(i * blk, blk)
        keysT[:, pl.ds(c0, blk)] = keys
        pmax[:, pl.ds(c0, blk)] = pm

    @pl.when(i == nblk)
    def _phase_select():
        keys = keysT[...]
        lo = jnp.full((e_dim, 1), _KEY_LO, jnp.int32)
        hi = jnp.full((e_dim, 1), _KEY_HI, jnp.int32)
        for _ in range(25):
            mid = (lo + hi + 1) // 2
            cnt = jnp.sum((keys >= mid).astype(jnp.int32), axis=1,
                          keepdims=True)
            ok = cnt >= cap
            lo = jnp.where(ok, mid, lo)
            hi = jnp.where(ok, hi, mid - 1)
        ks = jnp.where(lo == _KEY_LO, 0, lo)
        g_cnt = jnp.sum((keys > ks).astype(jnp.float32), axis=1, keepdims=True)
        thri[:, 0:1] = ks
        thrf[:, 0:1] = jnp.float32(cap) - g_cnt
        runeq[...] = jnp.zeros_like(runeq)
        runf[...] = jnp.zeros_like(runf)

    @pl.when(i > nblk)
    def _phase_dispatch():
        j = i - (nblk + 1)
        c0 = pl.multiple_of(j * cblk, cblk)
        kb = keysT[:, pl.ds(c0, cblk)]
        pm = pmax[:, pl.ds(c0, cblk)]
        ks = thri[:, 0:1]
        r_keep = thrf[:, 0:1]
        gt = kb > ks
        eq = kb == ks
        asg = kb > 0
        eqf = eq.astype(jnp.float32)
        p = eqf
        sh = 1
        while sh < cblk:
            p = p + jnp.pad(p, ((0, 0), (sh, 0)))[:, :cblk]
            sh *= 2
        prefix = p + runeq[:, 0:1]
        keep = eq & (prefix <= r_keep)
        disp = asg & (gt | keep)
        dispf = disp.astype(jnp.float32)
        runeq[:, 0:1] += jnp.sum(eqf, axis=1, keepdims=True)
        runf[:, 0:1] += jnp.sum(dispf, axis=1, keepdims=True)
        g = dispf * (pm / (pm + 1e-6))
        gates_ref[...] = g.T
        disp_ref[...] = dispf.T

        @pl.when(i == nblk + ncblk)
        def _():
            n_tokens = nblk * blk
            inv_n = jnp.float32(1.0 / n_tokens)
            p_mean = jnp.sum(pacc[...], axis=1, keepdims=True) * inv_n
            f_mean = runf[:, 0:1] * inv_n
            aux = 0.1 * jnp.sum(f_mean * p_mean) * e_dim
            z = 0.001 * jnp.sum(zacc[...]) * inv_n
            loss_ref[...] = jnp.reshape(aux + z, (1, 1))


def kernel(x, W_router):
    b_dim, t_dim, c_dim = x.shape
    e_dim = W_router.shape[0]
    n = b_dim * t_dim
    cap = int(1.25 * n / e_dim)
    blk = _BLK
    nblk = n // blk
    cblk = _CBLK
    ncblk = n // cblk
    x2 = x.reshape(n, c_dim)

    body = functools.partial(_router_body, cap, nblk, ncblk)
    gates, disp, loss = pl.pallas_call(
        body,
        grid=(nblk + 1 + ncblk,),
        in_specs=[
            pl.BlockSpec((blk, c_dim), lambda i: (jnp.minimum(i, nblk - 1), 0)),
            pl.BlockSpec((e_dim, c_dim), lambda i: (0, 0)),
        ],
        out_specs=[
            pl.BlockSpec((cblk, e_dim), lambda i: (jnp.maximum(i - (nblk + 1), 0), 0)),
            pl.BlockSpec((cblk, e_dim), lambda i: (jnp.maximum(i - (nblk + 1), 0), 0)),
            pl.BlockSpec((1, 1), lambda i: (0, 0)),
        ],
        out_shape=[
            jax.ShapeDtypeStruct((n, e_dim), jnp.float32),
            jax.ShapeDtypeStruct((n, e_dim), jnp.float32),
            jax.ShapeDtypeStruct((1, 1), jnp.float32),
        ],
        scratch_shapes=[
            pltpu.VMEM((e_dim, n), jnp.int32),
            pltpu.VMEM((1, n), jnp.float32),
            pltpu.VMEM((1, blk), jnp.float32),
            pltpu.VMEM((e_dim, blk), jnp.float32),
            pltpu.VMEM((e_dim, 128), jnp.int32),
            pltpu.VMEM((e_dim, 128), jnp.float32),
            pltpu.VMEM((e_dim, 128), jnp.float32),
            pltpu.VMEM((e_dim, 128), jnp.float32),
        ],
    )(x2, W_router)
    return (gates.reshape(b_dim, t_dim, e_dim),
            disp.reshape(b_dim, t_dim, e_dim),
            loss.reshape(()))


# speedup vs baseline: 1.0376x; 1.0376x over previous
import functools

import jax
import jax.numpy as jnp
from jax.experimental import pallas as pl
from jax.experimental.pallas import tpu as pltpu

_BLK = 1024
_CBLK = 8192
_KEY_LO = 0x3DFFFF00
_KEY_HI = 0x3F800000


def _router_body(cap, nblk, ncblk, x_ref, w_ref,
                 gates_ref, disp_ref, loss_ref,
                 keysT, pmax, zacc, pacc, thri, thrf, runeq, runf):
    i = pl.program_id(0)
    e_dim = w_ref.shape[0]
    blk = x_ref.shape[0]
    cblk = gates_ref.shape[0]

    @pl.when(i < nblk)
    def _phase_logits():
        xb = x_ref[...]
        w = w_ref[...]
        lg = jax.lax.dot_general(
            w, xb, (((1,), (1,)), ((), ())),
            preferred_element_type=jnp.float32)
        m = jnp.max(lg, axis=0, keepdims=True)
        ex = jnp.exp(lg - m)
        s = jnp.sum(ex, axis=0, keepdims=True)
        pr = ex / s
        lse = m + jnp.log(s)
        z_part = lse * lse

        @pl.when(i == 0)
        def _():
            zacc[...] = z_part
            pacc[...] = pr

        @pl.when(i > 0)
        def _():
            zacc[...] += z_part
            pacc[...] += pr

        pm = jnp.max(pr, axis=0, keepdims=True)
        eidx = jax.lax.broadcasted_iota(jnp.int32, (e_dim, blk), 0)
        cand = jnp.where(pr == pm, eidx, e_dim)
        amin = jnp.min(cand, axis=0, keepdims=True)
        kb = jax.lax.bitcast_convert_type(pr, jnp.int32)
        keys = jnp.where(eidx == amin, kb, 0)
        c0 = pl.multiple_of(i * blk, blk)
        keysT[:, pl.ds(c0, blk)] = keys
        pmax[:, pl.ds(c0, blk)] = pm

    @pl.when(i == nblk)
    def _phase_select():
        keys = keysT[...]
        lo = jnp.full((e_dim, 1), _KEY_LO, jnp.int32)
        hi = jnp.full((e_dim, 1), _KEY_HI, jnp.int32)
        for _ in range(25):
            mid = (lo + hi + 1) // 2
            cnt = jnp.sum((keys >= mid).astype(jnp.int32), axis=1,
                          keepdims=True)
            ok = cnt >= cap
            lo = jnp.where(ok, mid, lo)
            hi = jnp.where(ok, hi, mid - 1)
        ks = jnp.where(lo == _KEY_LO, 0, lo)
        g_cnt = jnp.sum((keys > ks).astype(jnp.float32), axis=1, keepdims=True)
        thri[:, 0:1] = ks
        thrf[:, 0:1] = jnp.float32(cap) - g_cnt
        runeq[...] = jnp.zeros_like(runeq)
        runf[...] = jnp.zeros_like(runf)

    @pl.when(i > nblk)
    def _phase_dispatch():
        j = i - (nblk + 1)
        c0 = pl.multiple_of(j * cblk, cblk)
        kb = keysT[:, pl.ds(c0, cblk)]
        pm = pmax[:, pl.ds(c0, cblk)]
        ks = thri[:, 0:1]
        r_keep = thrf[:, 0:1]
        gt = kb > ks
        eq = kb == ks
        asg = kb > 0
        eqf = eq.astype(jnp.float32)
        p = eqf
        sh = 1
        while sh < cblk:
            p = p + jnp.pad(p, ((0, 0), (sh, 0)))[:, :cblk]
            sh *= 2
        prefix = p + runeq[:, 0:1]
        keep = eq & (prefix <= r_keep)
        disp = asg & (gt | keep)
        dispf = disp.astype(jnp.float32)
        runeq[:, 0:1] += jnp.sum(eqf, axis=1, keepdims=True)
        runf[:, 0:1] += jnp.sum(dispf, axis=1, keepdims=True)
        g = dispf * (pm / (pm + 1e-6))
        gates_ref[...] = g.T
        disp_ref[...] = dispf.T

        @pl.when(i == nblk + ncblk)
        def _():
            n_tokens = nblk * blk
            inv_n = jnp.float32(1.0 / n_tokens)
            p_mean = jnp.sum(pacc[...], axis=1, keepdims=True) * inv_n
            f_mean = runf[:, 0:1] * inv_n
            aux = 0.1 * jnp.sum(f_mean * p_mean) * e_dim
            z = 0.001 * jnp.sum(zacc[...]) * inv_n
            loss_ref[...] = jnp.reshape(aux + z, (1, 1))


def kernel(x, W_router):
    b_dim, t_dim, c_dim = x.shape
    e_dim = W_router.shape[0]
    n = b_dim * t_dim
    cap = int(1.25 * n / e_dim)
    blk = _BLK
    nblk = n // blk
    cblk = _CBLK
    ncblk = n // cblk
    x2 = x.reshape(n, c_dim)

    body = functools.partial(_router_body, cap, nblk, ncblk)
    gates, disp, loss = pl.pallas_call(
        body,
        grid=(nblk + 1 + ncblk,),
        in_specs=[
            pl.BlockSpec((blk, c_dim), lambda i: (jnp.minimum(i, nblk - 1), 0)),
            pl.BlockSpec((e_dim, c_dim), lambda i: (0, 0)),
        ],
        out_specs=[
            pl.BlockSpec((cblk, e_dim), lambda i: (jnp.maximum(i - (nblk + 1), 0), 0)),
            pl.BlockSpec((cblk, e_dim), lambda i: (jnp.maximum(i - (nblk + 1), 0), 0)),
            pl.BlockSpec((1, 1), lambda i: (0, 0)),
        ],
        out_shape=[
            jax.ShapeDtypeStruct((n, e_dim), jnp.float32),
            jax.ShapeDtypeStruct((n, e_dim), jnp.float32),
            jax.ShapeDtypeStruct((1, 1), jnp.float32),
        ],
        scratch_shapes=[
            pltpu.VMEM((e_dim, n), jnp.int32),
            pltpu.VMEM((1, n), jnp.float32),
            pltpu.VMEM((1, blk), jnp.float32),
            pltpu.VMEM((e_dim, blk), jnp.float32),
            pltpu.VMEM((e_dim, 128), jnp.int32),
            pltpu.VMEM((e_dim, 128), jnp.float32),
            pltpu.VMEM((e_dim, 128), jnp.float32),
            pltpu.VMEM((e_dim, 128), jnp.float32),
        ],
    )(x2, W_router)
    return (gates.reshape(b_dim, t_dim, e_dim),
            disp.reshape(b_dim, t_dim, e_dim),
            loss.reshape(()))
